# baseline (speedup 1.0000x reference)
import functools

import jax
import jax.numpy as jnp
from jax import lax
from jax.experimental import pallas as pl
from jax.experimental.pallas import tpu as pltpu
from jax.experimental.pallas import tpu_sc as plsc

_DEPTH = 4
_K = 8192
_D = 256
_N = 32 * 576


_TM = 256
_GA = _N // _TM

_BOUNDS_B = (0, 2048, 4096, 6144, 8192)
_BOUNDS_C = (0, 1368, 2736, 4104, 5472, 6840, 8192)


def _make_amin_body(bounds):
    def body(r_ref, rn_ref, cbt_ref, cn_ref, idx_ref):
        r16 = r_ref[...].astype(jnp.bfloat16)
        mm = lax.dot_general(r16, cbt_ref[...], (((1,), (0,)), ((), ())),
                             preferred_element_type=jnp.float32)
        d = (rn_ref[...] + cn_ref[...]) - 2.0 * mm
        jj = lax.broadcasted_iota(jnp.int32, d.shape, 1)
        acc_v = None
        acc_i = None
        for s in range(len(bounds) - 1):
            lo, hi = bounds[s], bounds[s + 1]
            mask = (jj >= lo) & (jj < hi)
            dm = jnp.where(mask, d, jnp.inf)
            cm = jnp.min(dm, axis=1, keepdims=True)
            ci = jnp.min(jnp.where(dm == cm, jj, _K), axis=1)
            cmf = cm[:, 0]
            if acc_v is None:
                acc_v = cmf.astype(jnp.bfloat16).astype(jnp.float32)
                acc_i = ci
            else:
                take = cmf < acc_v
                acc_v = jnp.where(take, cmf, acc_v).astype(
                    jnp.bfloat16).astype(jnp.float32)
                acc_i = jnp.where(take, ci, acc_i)
        idx_ref[...] = acc_i
    return body


_amin_bodies = {"B": _make_amin_body(_BOUNDS_B), "C": _make_amin_body(_BOUNDS_C)}


def _argmin_call(r, rn, cbt16, cn, variant):
    return pl.pallas_call(
        _amin_bodies[variant],
        grid=(_GA,),
        in_specs=[
            pl.BlockSpec((_TM, _D), lambda i: (i, 0)),
            pl.BlockSpec((_TM, 1), lambda i: (i, 0)),
            pl.BlockSpec((_D, _K), lambda i: (0, 0)),
            pl.BlockSpec((1, _K), lambda i: (0, 0)),
        ],
        out_specs=pl.BlockSpec((_TM,), lambda i: (i,)),
        out_shape=jax.ShapeDtypeStruct((_N,), jnp.int32),
    )(r, rn, cbt16, cn)


_TE = 1024
_GE = _N // _TE


def _resid_body(rp_ref, q_ref, r_ref, ss_ref):
    r = rp_ref[...] - q_ref[...]
    r_ref[...] = r
    ss_ref[...] = jnp.full((1, 1, 128), jnp.sum(r * r), jnp.float32)


def _resid_call(rp, q):
    return pl.pallas_call(
        _resid_body,
        grid=(_GE,),
        in_specs=[
            pl.BlockSpec((_TE, _D), lambda i: (i, 0)),
            pl.BlockSpec((_TE, _D), lambda i: (i, 0)),
        ],
        out_specs=[
            pl.BlockSpec((_TE, _D), lambda i: (i, 0)),
            pl.BlockSpec((1, 1, 128), lambda i: (i, 0, 0)),
        ],
        out_shape=[
            jax.ShapeDtypeStruct((_N, _D), jnp.float32),
            jax.ShapeDtypeStruct((_GE, 1, 128), jnp.float32),
        ],
    )(rp, q)


def _final_body(x_ref, rp_ref, q_ref, out_ref, ss_ref):
    r = rp_ref[...] - q_ref[...]
    out_ref[...] = x_ref[...] - r
    ss_ref[...] = jnp.full((1, 1, 128), jnp.sum(r * r), jnp.float32)


def _final_call(x, rp, q):
    return pl.pallas_call(
        _final_body,
        grid=(_GE,),
        in_specs=[
            pl.BlockSpec((_TE, _D), lambda i: (i, 0)),
            pl.BlockSpec((_TE, _D), lambda i: (i, 0)),
            pl.BlockSpec((_TE, _D), lambda i: (i, 0)),
        ],
        out_specs=[
            pl.BlockSpec((_TE, _D), lambda i: (i, 0)),
            pl.BlockSpec((1, 1, 128), lambda i: (i, 0, 0)),
        ],
        out_shape=[
            jax.ShapeDtypeStruct((_N, _D), jnp.float32),
            jax.ShapeDtypeStruct((_GE, 1, 128), jnp.float32),
        ],
    )(x, rp, q)


_NC, _NS = 2, 16
_NW = _NC * _NS
_BPW = _N // _NW
_CH = 96
_NCH = _BPW // _CH


def _make_gather():
    mesh = plsc.VectorSubcoreMesh(core_axis_name="c", subcore_axis_name="s")

    @functools.partial(
        pl.kernel,
        mesh=mesh,
        out_type=jax.ShapeDtypeStruct((_N, _D), jnp.float32),
        scratch_types=[
            pltpu.VMEM((_BPW,), jnp.int32),
            pltpu.VMEM((_CH, _D), jnp.float32),
            pltpu.VMEM((_CH, _D), jnp.float32),
            pltpu.SemaphoreType.DMA,
            pltpu.SemaphoreType.DMA,
        ],
    )
    def gather(cb_hbm, idx_hbm, out_hbm, idx_v, buf0, buf1, sem0, sem1):
        wid = lax.axis_index("s") * _NC + lax.axis_index("c")
        base = wid * _BPW
        pltpu.sync_copy(idx_hbm.at[pl.ds(base, _BPW)], idx_v)
        bufs = (buf0, buf1)
        sems = (sem0, sem1)
        cps = []
        for c in range(_NCH):
            cps.append(pltpu.async_copy(
                cb_hbm.at[idx_v.at[pl.ds(c * _CH, _CH)]],
                bufs[c % 2], sems[c % 2]))
            if c >= 1:
                cps[c - 1].wait()
                pltpu.sync_copy(bufs[(c - 1) % 2],
                                out_hbm.at[pl.ds(base + (c - 1) * _CH, _CH)])
        cps[-1].wait()
        pltpu.sync_copy(bufs[(_NCH - 1) % 2],
                        out_hbm.at[pl.ds(base + (_NCH - 1) * _CH, _CH)])

    return gather


_gather_rows = _make_gather()


def kernel(x, codebooks):
    b, t, d = x.shape
    xf = x.reshape(_N, _D)
    cbt16 = jnp.transpose(codebooks, (0, 2, 1)).astype(jnp.bfloat16)

    r = xf
    idx_list = []
    ss_list = []
    quants = None
    for i in range(_DEPTH):
        rn = jnp.sum(r ** 2.0, axis=1, keepdims=True)
        cn = jnp.sum(codebooks[i] ** 2.0, axis=1)[None, :]
        idx = _argmin_call(r, rn, cbt16[i], cn,
                           "C" if i == 3 else "B")
        q = _gather_rows(codebooks[i], idx)
        idx_list.append(idx)
        if i < _DEPTH - 1:
            r, ss = _resid_call(r, q)
        else:
            quants, ss = _final_call(xf, r, q)
        ss_list.append(ss)

    denom = jnp.float32(_N * _D)
    losses = [jnp.sum(s[:, 0, 0]) / denom for s in ss_list]
    commitment_loss = jnp.mean(jnp.stack(losses))
    codes = jnp.stack(idx_list, axis=-1).reshape(b, t, _DEPTH)
    quants = quants.reshape(b, t, d)
    return quants, commitment_loss, codes


# speedup vs baseline: 1.2090x; 1.2090x over previous
import functools

import jax
import jax.numpy as jnp
from jax import lax
from jax.experimental import pallas as pl
from jax.experimental.pallas import tpu as pltpu
from jax.experimental.pallas import tpu_sc as plsc

_DEPTH = 4
_K = 8192
_D = 256
_N = 32 * 576


_TM = 256
_GA = _N // _TM

_BOUNDS_B = (0, 2048, 4096, 6144, 8192)
_BOUNDS_C = (0, 1368, 2736, 4104, 5472, 6840, 8192)


def _seg_min(d, lo, hi):
    inf = jnp.float32(jnp.inf)
    tm = d.shape[0]
    lo_a = -(-lo // 128) * 128
    hi_a = (hi // 128) * 128
    parts = []
    if lo_a > lo:
        b0 = lo_a - 128
        blk = d[:, b0:lo_a]
        jb = lax.broadcasted_iota(jnp.int32, (tm, 128), 1) + b0
        parts.append(jnp.min(jnp.where(jb >= lo, blk, inf), axis=1))
    if hi_a > lo_a:
        parts.append(jnp.min(d[:, lo_a:hi_a], axis=1))
    if hi > hi_a:
        blk = d[:, hi_a:hi_a + 128]
        jb = lax.broadcasted_iota(jnp.int32, (tm, 128), 1) + hi_a
        parts.append(jnp.min(jnp.where(jb < hi, blk, inf), axis=1))
    cm = parts[0]
    for p in parts[1:]:
        cm = jnp.minimum(cm, p)
    return cm


def _make_amin_body(bounds):
    def body(r_ref, rn_ref, cbt_ref, cn_ref, idx_ref):
        r16 = r_ref[...].astype(jnp.bfloat16)
        mm = lax.dot_general(r16, cbt_ref[...], (((1,), (0,)), ((), ())),
                             preferred_element_type=jnp.float32)
        d = (rn_ref[...] + cn_ref[...]) - 2.0 * mm
        acc_q = acc_cm = acc_lo = acc_hi = None
        for s in range(len(bounds) - 1):
            lo, hi = bounds[s], bounds[s + 1]
            cm = _seg_min(d, lo, hi)
            lo_v = jnp.full(cm.shape, lo, jnp.int32)
            hi_v = jnp.full(cm.shape, hi, jnp.int32)
            if acc_q is None:
                acc_q = cm.astype(jnp.bfloat16).astype(jnp.float32)
                acc_cm, acc_lo, acc_hi = cm, lo_v, hi_v
            else:
                take = cm < acc_q
                acc_q = jnp.where(take, cm, acc_q).astype(
                    jnp.bfloat16).astype(jnp.float32)
                acc_cm = jnp.where(take, cm, acc_cm)
                acc_lo = jnp.where(take, lo_v, acc_lo)
                acc_hi = jnp.where(take, hi_v, acc_hi)
        jj = lax.broadcasted_iota(jnp.int32, d.shape, 1)
        sel = ((d == acc_cm[:, None]) & (jj >= acc_lo[:, None])
               & (jj < acc_hi[:, None]))
        idx_ref[...] = jnp.min(jnp.where(sel, jj, _K), axis=1)
    return body


_amin_bodies = {"B": _make_amin_body(_BOUNDS_B), "C": _make_amin_body(_BOUNDS_C)}


def _argmin_call(r, rn, cbt16, cn, variant):
    return pl.pallas_call(
        _amin_bodies[variant],
        grid=(_GA,),
        in_specs=[
            pl.BlockSpec((_TM, _D), lambda i: (i, 0)),
            pl.BlockSpec((_TM, 1), lambda i: (i, 0)),
            pl.BlockSpec((_D, _K), lambda i: (0, 0)),
            pl.BlockSpec((1, _K), lambda i: (0, 0)),
        ],
        out_specs=pl.BlockSpec((_TM,), lambda i: (i,)),
        out_shape=jax.ShapeDtypeStruct((_N,), jnp.int32),
    )(r, rn, cbt16, cn)


_TE = 1024
_GE = _N // _TE


def _resid_body(rp_ref, q_ref, r_ref, ss_ref):
    r = rp_ref[...] - q_ref[...]
    r_ref[...] = r
    ss_ref[...] = jnp.full((1, 1, 128), jnp.sum(r * r), jnp.float32)


def _resid_call(rp, q):
    return pl.pallas_call(
        _resid_body,
        grid=(_GE,),
        in_specs=[
            pl.BlockSpec((_TE, _D), lambda i: (i, 0)),
            pl.BlockSpec((_TE, _D), lambda i: (i, 0)),
        ],
        out_specs=[
            pl.BlockSpec((_TE, _D), lambda i: (i, 0)),
            pl.BlockSpec((1, 1, 128), lambda i: (i, 0, 0)),
        ],
        out_shape=[
            jax.ShapeDtypeStruct((_N, _D), jnp.float32),
            jax.ShapeDtypeStruct((_GE, 1, 128), jnp.float32),
        ],
    )(rp, q)


def _final_body(x_ref, rp_ref, q_ref, out_ref, ss_ref):
    r = rp_ref[...] - q_ref[...]
    out_ref[...] = x_ref[...] - r
    ss_ref[...] = jnp.full((1, 1, 128), jnp.sum(r * r), jnp.float32)


def _final_call(x, rp, q):
    return pl.pallas_call(
        _final_body,
        grid=(_GE,),
        in_specs=[
            pl.BlockSpec((_TE, _D), lambda i: (i, 0)),
            pl.BlockSpec((_TE, _D), lambda i: (i, 0)),
            pl.BlockSpec((_TE, _D), lambda i: (i, 0)),
        ],
        out_specs=[
            pl.BlockSpec((_TE, _D), lambda i: (i, 0)),
            pl.BlockSpec((1, 1, 128), lambda i: (i, 0, 0)),
        ],
        out_shape=[
            jax.ShapeDtypeStruct((_N, _D), jnp.float32),
            jax.ShapeDtypeStruct((_GE, 1, 128), jnp.float32),
        ],
    )(x, rp, q)


_NC, _NS = 2, 16
_NW = _NC * _NS
_BPW = _N // _NW
_CH = 96
_NCH = _BPW // _CH


def _make_gather():
    mesh = plsc.VectorSubcoreMesh(core_axis_name="c", subcore_axis_name="s")

    @functools.partial(
        pl.kernel,
        mesh=mesh,
        out_type=jax.ShapeDtypeStruct((_N, _D), jnp.float32),
        scratch_types=[
            pltpu.VMEM((_BPW,), jnp.int32),
            pltpu.VMEM((_CH, _D), jnp.float32),
            pltpu.VMEM((_CH, _D), jnp.float32),
            pltpu.SemaphoreType.DMA,
            pltpu.SemaphoreType.DMA,
        ],
    )
    def gather(cb_hbm, idx_hbm, out_hbm, idx_v, buf0, buf1, sem0, sem1):
        wid = lax.axis_index("s") * _NC + lax.axis_index("c")
        base = wid * _BPW
        pltpu.sync_copy(idx_hbm.at[pl.ds(base, _BPW)], idx_v)
        bufs = (buf0, buf1)
        sems = (sem0, sem1)
        cps = []
        for c in range(_NCH):
            cps.append(pltpu.async_copy(
                cb_hbm.at[idx_v.at[pl.ds(c * _CH, _CH)]],
                bufs[c % 2], sems[c % 2]))
            if c >= 1:
                cps[c - 1].wait()
                pltpu.sync_copy(bufs[(c - 1) % 2],
                                out_hbm.at[pl.ds(base + (c - 1) * _CH, _CH)])
        cps[-1].wait()
        pltpu.sync_copy(bufs[(_NCH - 1) % 2],
                        out_hbm.at[pl.ds(base + (_NCH - 1) * _CH, _CH)])

    return gather


_gather_rows = _make_gather()


def kernel(x, codebooks):
    b, t, d = x.shape
    xf = x.reshape(_N, _D)
    cbt16 = jnp.transpose(codebooks, (0, 2, 1)).astype(jnp.bfloat16)

    r = xf
    idx_list = []
    ss_list = []
    quants = None
    for i in range(_DEPTH):
        rn = jnp.sum(r ** 2.0, axis=1, keepdims=True)
        cn = jnp.sum(codebooks[i] ** 2.0, axis=1)[None, :]
        idx = _argmin_call(r, rn, cbt16[i], cn,
                           "C" if i == 3 else "B")
        q = _gather_rows(codebooks[i], idx)
        idx_list.append(idx)
        if i < _DEPTH - 1:
            r, ss = _resid_call(r, q)
        else:
            quants, ss = _final_call(xf, r, q)
        ss_list.append(ss)

    denom = jnp.float32(_N * _D)
    losses = [jnp.sum(s[:, 0, 0]) / denom for s in ss_list]
    commitment_loss = jnp.mean(jnp.stack(losses))
    codes = jnp.stack(idx_list, axis=-1).reshape(b, t, _DEPTH)
    quants = quants.reshape(b, t, d)
    return quants, commitment_loss, codes


# speedup vs baseline: 1.6035x; 1.3263x over previous
import functools

import jax
import jax.numpy as jnp
from jax import lax
from jax.experimental import pallas as pl
from jax.experimental.pallas import tpu as pltpu
from jax.experimental.pallas import tpu_sc as plsc

_DEPTH = 4
_K = 8192
_D = 256
_N = 32 * 576


_TM = 256
_GA = _N // _TM

_BOUNDS_B = (0, 2048, 4096, 6144, 8192)
_BOUNDS_C = (0, 1368, 2736, 4104, 5472, 6840, 8192)


def _seg_parts(d, lo, hi):
    tm = d.shape[0]
    lo_a = -(-lo // 128) * 128
    hi_a = (hi // 128) * 128
    parts = []
    if lo_a > lo:
        b0 = lo_a - 128
        parts.append((d[:, b0:lo_a], b0, lo, min(hi, lo_a)))
    if hi_a > lo_a:
        parts.append((d[:, lo_a:hi_a], lo_a, None, None))
    if hi > hi_a:
        parts.append((d[:, hi_a:hi_a + 128], hi_a, max(lo, hi_a), hi))
    return parts


def _make_amin_body(bounds):
    def body(r_ref, rn_ref, cbt_ref, cn_ref, idx_ref):
        r16 = r_ref[...].astype(jnp.bfloat16)
        mm = lax.dot_general(r16, cbt_ref[...], (((1,), (0,)), ((), ())),
                             preferred_element_type=jnp.float32)
        d = (rn_ref[...] + cn_ref[...]) - 2.0 * mm
        inf = jnp.float32(jnp.inf)
        acc_q = acc_i = None
        for s in range(len(bounds) - 1):
            lo, hi = bounds[s], bounds[s + 1]
            parts = _seg_parts(d, lo, hi)
            masked = []
            for blk, off, cl, ch in parts:
                if cl is not None:
                    jb = lax.broadcasted_iota(jnp.int32, blk.shape, 1) + off
                    blk = jnp.where((jb >= cl) & (jb < ch), blk, inf)
                masked.append((blk, off))
            cm = None
            for blk, off in masked:
                pm = jnp.min(blk, axis=1)
                cm = pm if cm is None else jnp.minimum(cm, pm)
            ci = None
            cmk = cm[:, None]
            for blk, off in masked:
                jb = lax.broadcasted_iota(jnp.int32, blk.shape, 1) + off
                pi = jnp.min(jnp.where(blk == cmk, jb, _K), axis=1)
                ci = pi if ci is None else jnp.minimum(ci, pi)
            if acc_q is None:
                acc_q = cm.astype(jnp.bfloat16).astype(jnp.float32)
                acc_i = ci
            else:
                take = cm < acc_q
                acc_q = jnp.where(take, cm, acc_q).astype(
                    jnp.bfloat16).astype(jnp.float32)
                acc_i = jnp.where(take, ci, acc_i)
        idx_ref[...] = acc_i
    return body


_amin_bodies = {"B": _make_amin_body(_BOUNDS_B), "C": _make_amin_body(_BOUNDS_C)}


def _argmin_call(r, rn, cbt16, cn, variant):
    return pl.pallas_call(
        _amin_bodies[variant],
        grid=(_GA,),
        in_specs=[
            pl.BlockSpec((_TM, _D), lambda i: (i, 0)),
            pl.BlockSpec((_TM, 1), lambda i: (i, 0)),
            pl.BlockSpec((_D, _K), lambda i: (0, 0)),
            pl.BlockSpec((1, _K), lambda i: (0, 0)),
        ],
        out_specs=pl.BlockSpec((_TM,), lambda i: (i,)),
        out_shape=jax.ShapeDtypeStruct((_N,), jnp.int32),
    )(r, rn, cbt16, cn)


_TE = 1024
_GE = _N // _TE


def _resid_body(rp_ref, q_ref, r_ref, ss_ref):
    r = rp_ref[...] - q_ref[...]
    r_ref[...] = r
    ss_ref[...] = jnp.full((1, 1, 128), jnp.sum(r * r), jnp.float32)


def _resid_call(rp, q):
    return pl.pallas_call(
        _resid_body,
        grid=(_GE,),
        in_specs=[
            pl.BlockSpec((_TE, _D), lambda i: (i, 0)),
            pl.BlockSpec((_TE, _D), lambda i: (i, 0)),
        ],
        out_specs=[
            pl.BlockSpec((_TE, _D), lambda i: (i, 0)),
            pl.BlockSpec((1, 1, 128), lambda i: (i, 0, 0)),
        ],
        out_shape=[
            jax.ShapeDtypeStruct((_N, _D), jnp.float32),
            jax.ShapeDtypeStruct((_GE, 1, 128), jnp.float32),
        ],
    )(rp, q)


def _final_body(x_ref, rp_ref, q_ref, out_ref, ss_ref):
    r = rp_ref[...] - q_ref[...]
    out_ref[...] = x_ref[...] - r
    ss_ref[...] = jnp.full((1, 1, 128), jnp.sum(r * r), jnp.float32)


def _final_call(x, rp, q):
    return pl.pallas_call(
        _final_body,
        grid=(_GE,),
        in_specs=[
            pl.BlockSpec((_TE, _D), lambda i: (i, 0)),
            pl.BlockSpec((_TE, _D), lambda i: (i, 0)),
            pl.BlockSpec((_TE, _D), lambda i: (i, 0)),
        ],
        out_specs=[
            pl.BlockSpec((_TE, _D), lambda i: (i, 0)),
            pl.BlockSpec((1, 1, 128), lambda i: (i, 0, 0)),
        ],
        out_shape=[
            jax.ShapeDtypeStruct((_N, _D), jnp.float32),
            jax.ShapeDtypeStruct((_GE, 1, 128), jnp.float32),
        ],
    )(x, rp, q)


_NC, _NS = 2, 16
_NW = _NC * _NS
_BPW = _N // _NW
_CH = 96
_NCH = _BPW // _CH


def _make_gather():
    mesh = plsc.VectorSubcoreMesh(core_axis_name="c", subcore_axis_name="s")

    @functools.partial(
        pl.kernel,
        mesh=mesh,
        out_type=jax.ShapeDtypeStruct((_N, _D), jnp.float32),
        scratch_types=[
            pltpu.VMEM((_BPW,), jnp.int32),
            pltpu.VMEM((_CH, _D), jnp.float32),
            pltpu.VMEM((_CH, _D), jnp.float32),
            pltpu.SemaphoreType.DMA,
            pltpu.SemaphoreType.DMA,
        ],
    )
    def gather(cb_hbm, idx_hbm, out_hbm, idx_v, buf0, buf1, sem0, sem1):
        wid = lax.axis_index("s") * _NC + lax.axis_index("c")
        base = wid * _BPW
        pltpu.sync_copy(idx_hbm.at[pl.ds(base, _BPW)], idx_v)
        bufs = (buf0, buf1)
        sems = (sem0, sem1)
        cps = []
        for c in range(_NCH):
            cps.append(pltpu.async_copy(
                cb_hbm.at[idx_v.at[pl.ds(c * _CH, _CH)]],
                bufs[c % 2], sems[c % 2]))
            if c >= 1:
                cps[c - 1].wait()
                pltpu.sync_copy(bufs[(c - 1) % 2],
                                out_hbm.at[pl.ds(base + (c - 1) * _CH, _CH)])
        cps[-1].wait()
        pltpu.sync_copy(bufs[(_NCH - 1) % 2],
                        out_hbm.at[pl.ds(base + (_NCH - 1) * _CH, _CH)])

    return gather


_gather_rows = _make_gather()


def kernel(x, codebooks):
    b, t, d = x.shape
    xf = x.reshape(_N, _D)
    cbt16 = jnp.transpose(codebooks, (0, 2, 1)).astype(jnp.bfloat16)

    r = xf
    idx_list = []
    ss_list = []
    quants = None
    for i in range(_DEPTH):
        rn = jnp.sum(r ** 2.0, axis=1, keepdims=True)
        cn = jnp.sum(codebooks[i] ** 2.0, axis=1)[None, :]
        idx = _argmin_call(r, rn, cbt16[i], cn,
                           "C" if i == 3 else "B")
        q = _gather_rows(codebooks[i], idx)
        idx_list.append(idx)
        if i < _DEPTH - 1:
            r, ss = _resid_call(r, q)
        else:
            quants, ss = _final_call(xf, r, q)
        ss_list.append(ss)

    denom = jnp.float32(_N * _D)
    losses = [jnp.sum(s[:, 0, 0]) / denom for s in ss_list]
    commitment_loss = jnp.mean(jnp.stack(losses))
    codes = jnp.stack(idx_list, axis=-1).reshape(b, t, _DEPTH)
    quants = quants.reshape(b, t, d)
    return quants, commitment_loss, codes


# speedup vs baseline: 1.7126x; 1.0681x over previous
import functools

import jax
import jax.numpy as jnp
from jax import lax
from jax.experimental import pallas as pl
from jax.experimental.pallas import tpu as pltpu
from jax.experimental.pallas import tpu_sc as plsc

_DEPTH = 4
_K = 8192
_D = 256
_N = 32 * 576


_TM = 512
_GA = _N // _TM

_BOUNDS_B = (0, 2048, 4096, 6144, 8192)
_BOUNDS_C = (0, 1368, 2736, 4104, 5472, 6840, 8192)


def _seg_parts(d, lo, hi):
    tm = d.shape[0]
    lo_a = -(-lo // 128) * 128
    hi_a = (hi // 128) * 128
    parts = []
    if lo_a > lo:
        b0 = lo_a - 128
        parts.append((d[:, b0:lo_a], b0, lo, min(hi, lo_a)))
    if hi_a > lo_a:
        parts.append((d[:, lo_a:hi_a], lo_a, None, None))
    if hi > hi_a:
        parts.append((d[:, hi_a:hi_a + 128], hi_a, max(lo, hi_a), hi))
    return parts


def _make_amin_body(bounds):
    def body(r_ref, rn_ref, cbt_ref, cn_ref, idx_ref):
        r16 = r_ref[...].astype(jnp.bfloat16)
        mm = lax.dot_general(r16, cbt_ref[...], (((1,), (0,)), ((), ())),
                             preferred_element_type=jnp.float32)
        d = (rn_ref[...] + cn_ref[...]) - 2.0 * mm
        inf = jnp.float32(jnp.inf)
        acc_q = acc_i = None
        for s in range(len(bounds) - 1):
            lo, hi = bounds[s], bounds[s + 1]
            parts = _seg_parts(d, lo, hi)
            masked = []
            for blk, off, cl, ch in parts:
                if cl is not None:
                    jb = lax.broadcasted_iota(jnp.int32, blk.shape, 1) + off
                    blk = jnp.where((jb >= cl) & (jb < ch), blk, inf)
                masked.append((blk, off))
            cm = None
            for blk, off in masked:
                pm = jnp.min(blk, axis=1)
                cm = pm if cm is None else jnp.minimum(cm, pm)
            ci = None
            cmk = cm[:, None]
            for blk, off in masked:
                jb = lax.broadcasted_iota(jnp.int32, blk.shape, 1) + off
                pi = jnp.min(jnp.where(blk == cmk, jb, _K), axis=1)
                ci = pi if ci is None else jnp.minimum(ci, pi)
            if acc_q is None:
                acc_q = cm.astype(jnp.bfloat16).astype(jnp.float32)
                acc_i = ci
            else:
                take = cm < acc_q
                acc_q = jnp.where(take, cm, acc_q).astype(
                    jnp.bfloat16).astype(jnp.float32)
                acc_i = jnp.where(take, ci, acc_i)
        idx_ref[...] = acc_i
    return body


_amin_bodies = {"B": _make_amin_body(_BOUNDS_B), "C": _make_amin_body(_BOUNDS_C)}


def _argmin_call(r, rn, cbt16, cn, variant):
    return pl.pallas_call(
        _amin_bodies[variant],
        grid=(_GA,),
        in_specs=[
            pl.BlockSpec((_TM, _D), lambda i: (i, 0)),
            pl.BlockSpec((_TM, 1), lambda i: (i, 0)),
            pl.BlockSpec((_D, _K), lambda i: (0, 0)),
            pl.BlockSpec((1, _K), lambda i: (0, 0)),
        ],
        out_specs=pl.BlockSpec((_TM,), lambda i: (i,)),
        out_shape=jax.ShapeDtypeStruct((_N,), jnp.int32),
    )(r, rn, cbt16, cn)


_TE = 1024
_GE = _N // _TE


def _resid_body(rp_ref, q_ref, r_ref, ss_ref):
    r = rp_ref[...] - q_ref[...]
    r_ref[...] = r
    ss_ref[...] = jnp.full((1, 1, 128), jnp.sum(r * r), jnp.float32)


def _resid_call(rp, q):
    return pl.pallas_call(
        _resid_body,
        grid=(_GE,),
        in_specs=[
            pl.BlockSpec((_TE, _D), lambda i: (i, 0)),
            pl.BlockSpec((_TE, _D), lambda i: (i, 0)),
        ],
        out_specs=[
            pl.BlockSpec((_TE, _D), lambda i: (i, 0)),
            pl.BlockSpec((1, 1, 128), lambda i: (i, 0, 0)),
        ],
        out_shape=[
            jax.ShapeDtypeStruct((_N, _D), jnp.float32),
            jax.ShapeDtypeStruct((_GE, 1, 128), jnp.float32),
        ],
    )(rp, q)


def _final_body(x_ref, rp_ref, q_ref, out_ref, ss_ref):
    r = rp_ref[...] - q_ref[...]
    out_ref[...] = x_ref[...] - r
    ss_ref[...] = jnp.full((1, 1, 128), jnp.sum(r * r), jnp.float32)


def _final_call(x, rp, q):
    return pl.pallas_call(
        _final_body,
        grid=(_GE,),
        in_specs=[
            pl.BlockSpec((_TE, _D), lambda i: (i, 0)),
            pl.BlockSpec((_TE, _D), lambda i: (i, 0)),
            pl.BlockSpec((_TE, _D), lambda i: (i, 0)),
        ],
        out_specs=[
            pl.BlockSpec((_TE, _D), lambda i: (i, 0)),
            pl.BlockSpec((1, 1, 128), lambda i: (i, 0, 0)),
        ],
        out_shape=[
            jax.ShapeDtypeStruct((_N, _D), jnp.float32),
            jax.ShapeDtypeStruct((_GE, 1, 128), jnp.float32),
        ],
    )(x, rp, q)


_NC, _NS = 2, 16
_NW = _NC * _NS
_BPW = _N // _NW
_CH = 96
_NCH = _BPW // _CH


def _make_gather():
    mesh = plsc.VectorSubcoreMesh(core_axis_name="c", subcore_axis_name="s")

    @functools.partial(
        pl.kernel,
        mesh=mesh,
        out_type=jax.ShapeDtypeStruct((_N, _D), jnp.float32),
        scratch_types=[
            pltpu.VMEM((_BPW,), jnp.int32),
            pltpu.VMEM((_CH, _D), jnp.float32),
            pltpu.VMEM((_CH, _D), jnp.float32),
            pltpu.SemaphoreType.DMA,
            pltpu.SemaphoreType.DMA,
        ],
    )
    def gather(cb_hbm, idx_hbm, out_hbm, idx_v, buf0, buf1, sem0, sem1):
        wid = lax.axis_index("s") * _NC + lax.axis_index("c")
        base = wid * _BPW
        pltpu.sync_copy(idx_hbm.at[pl.ds(base, _BPW)], idx_v)
        bufs = (buf0, buf1)
        sems = (sem0, sem1)
        cps = []
        for c in range(_NCH):
            cps.append(pltpu.async_copy(
                cb_hbm.at[idx_v.at[pl.ds(c * _CH, _CH)]],
                bufs[c % 2], sems[c % 2]))
            if c >= 1:
                cps[c - 1].wait()
                pltpu.sync_copy(bufs[(c - 1) % 2],
                                out_hbm.at[pl.ds(base + (c - 1) * _CH, _CH)])
        cps[-1].wait()
        pltpu.sync_copy(bufs[(_NCH - 1) % 2],
                        out_hbm.at[pl.ds(base + (_NCH - 1) * _CH, _CH)])

    return gather


_gather_rows = _make_gather()


def kernel(x, codebooks):
    b, t, d = x.shape
    xf = x.reshape(_N, _D)
    cbt16 = jnp.transpose(codebooks, (0, 2, 1)).astype(jnp.bfloat16)

    r = xf
    idx_list = []
    ss_list = []
    quants = None
    for i in range(_DEPTH):
        rn = jnp.sum(r ** 2.0, axis=1, keepdims=True)
        cn = jnp.sum(codebooks[i] ** 2.0, axis=1)[None, :]
        idx = _argmin_call(r, rn, cbt16[i], cn,
                           "C" if i == 3 else "B")
        q = _gather_rows(codebooks[i], idx)
        idx_list.append(idx)
        if i < _DEPTH - 1:
            r, ss = _resid_call(r, q)
        else:
            quants, ss = _final_call(xf, r, q)
        ss_list.append(ss)

    denom = jnp.float32(_N * _D)
    losses = [jnp.sum(s[:, 0, 0]) / denom for s in ss_list]
    commitment_loss = jnp.mean(jnp.stack(losses))
    codes = jnp.stack(idx_list, axis=-1).reshape(b, t, _DEPTH)
    quants = quants.reshape(b, t, d)
    return quants, commitment_loss, codes


# speedup vs baseline: 1.7921x; 1.0464x over previous
import functools

import jax
import jax.numpy as jnp
from jax import lax
from jax.experimental import pallas as pl
from jax.experimental.pallas import tpu as pltpu
from jax.experimental.pallas import tpu_sc as plsc

_DEPTH = 4
_K = 8192
_D = 256
_N = 32 * 576


_TM = 1024
_GA = _N // _TM

_BOUNDS_B = (0, 2048, 4096, 6144, 8192)
_BOUNDS_C = (0, 1368, 2736, 4104, 5472, 6840, 8192)


def _seg_parts(d, lo, hi):
    tm = d.shape[0]
    lo_a = -(-lo // 128) * 128
    hi_a = (hi // 128) * 128
    parts = []
    if lo_a > lo:
        b0 = lo_a - 128
        parts.append((d[:, b0:lo_a], b0, lo, min(hi, lo_a)))
    if hi_a > lo_a:
        parts.append((d[:, lo_a:hi_a], lo_a, None, None))
    if hi > hi_a:
        parts.append((d[:, hi_a:hi_a + 128], hi_a, max(lo, hi_a), hi))
    return parts


def _make_amin_body(bounds):
    def body(r_ref, rn_ref, cbt_ref, cn_ref, idx_ref):
        r16 = r_ref[...].astype(jnp.bfloat16)
        mm = lax.dot_general(r16, cbt_ref[...], (((1,), (0,)), ((), ())),
                             preferred_element_type=jnp.float32)
        d = (rn_ref[...] + cn_ref[...]) - 2.0 * mm
        inf = jnp.float32(jnp.inf)
        acc_q = acc_i = None
        for s in range(len(bounds) - 1):
            lo, hi = bounds[s], bounds[s + 1]
            parts = _seg_parts(d, lo, hi)
            masked = []
            for blk, off, cl, ch in parts:
                if cl is not None:
                    jb = lax.broadcasted_iota(jnp.int32, blk.shape, 1) + off
                    blk = jnp.where((jb >= cl) & (jb < ch), blk, inf)
                masked.append((blk, off))
            cm = None
            for blk, off in masked:
                pm = jnp.min(blk, axis=1)
                cm = pm if cm is None else jnp.minimum(cm, pm)
            ci = None
            cmk = cm[:, None]
            for blk, off in masked:
                jb = lax.broadcasted_iota(jnp.int32, blk.shape, 1) + off
                pi = jnp.min(jnp.where(blk == cmk, jb, _K), axis=1)
                ci = pi if ci is None else jnp.minimum(ci, pi)
            if acc_q is None:
                acc_q = cm.astype(jnp.bfloat16).astype(jnp.float32)
                acc_i = ci
            else:
                take = cm < acc_q
                acc_q = jnp.where(take, cm, acc_q).astype(
                    jnp.bfloat16).astype(jnp.float32)
                acc_i = jnp.where(take, ci, acc_i)
        idx_ref[...] = acc_i
    return body


_amin_bodies = {"B": _make_amin_body(_BOUNDS_B), "C": _make_amin_body(_BOUNDS_C)}


def _argmin_call(r, rn, cbt16, cn, variant):
    return pl.pallas_call(
        _amin_bodies[variant],
        grid=(_GA,),
        in_specs=[
            pl.BlockSpec((_TM, _D), lambda i: (i, 0)),
            pl.BlockSpec((_TM, 1), lambda i: (i, 0)),
            pl.BlockSpec((_D, _K), lambda i: (0, 0)),
            pl.BlockSpec((1, _K), lambda i: (0, 0)),
        ],
        out_specs=pl.BlockSpec((_TM,), lambda i: (i,)),
        out_shape=jax.ShapeDtypeStruct((_N,), jnp.int32),
    )(r, rn, cbt16, cn)


_TE = 1024
_GE = _N // _TE


def _resid_body(rp_ref, q_ref, r_ref, ss_ref):
    r = rp_ref[...] - q_ref[...]
    r_ref[...] = r
    ss_ref[...] = jnp.full((1, 1, 128), jnp.sum(r * r), jnp.float32)


def _resid_call(rp, q):
    return pl.pallas_call(
        _resid_body,
        grid=(_GE,),
        in_specs=[
            pl.BlockSpec((_TE, _D), lambda i: (i, 0)),
            pl.BlockSpec((_TE, _D), lambda i: (i, 0)),
        ],
        out_specs=[
            pl.BlockSpec((_TE, _D), lambda i: (i, 0)),
            pl.BlockSpec((1, 1, 128), lambda i: (i, 0, 0)),
        ],
        out_shape=[
            jax.ShapeDtypeStruct((_N, _D), jnp.float32),
            jax.ShapeDtypeStruct((_GE, 1, 128), jnp.float32),
        ],
    )(rp, q)


def _final_body(x_ref, rp_ref, q_ref, out_ref, ss_ref):
    r = rp_ref[...] - q_ref[...]
    out_ref[...] = x_ref[...] - r
    ss_ref[...] = jnp.full((1, 1, 128), jnp.sum(r * r), jnp.float32)


def _final_call(x, rp, q):
    return pl.pallas_call(
        _final_body,
        grid=(_GE,),
        in_specs=[
            pl.BlockSpec((_TE, _D), lambda i: (i, 0)),
            pl.BlockSpec((_TE, _D), lambda i: (i, 0)),
            pl.BlockSpec((_TE, _D), lambda i: (i, 0)),
        ],
        out_specs=[
            pl.BlockSpec((_TE, _D), lambda i: (i, 0)),
            pl.BlockSpec((1, 1, 128), lambda i: (i, 0, 0)),
        ],
        out_shape=[
            jax.ShapeDtypeStruct((_N, _D), jnp.float32),
            jax.ShapeDtypeStruct((_GE, 1, 128), jnp.float32),
        ],
    )(x, rp, q)


_NC, _NS = 2, 16
_NW = _NC * _NS
_BPW = _N // _NW
_CH = 96
_NCH = _BPW // _CH


def _make_gather():
    mesh = plsc.VectorSubcoreMesh(core_axis_name="c", subcore_axis_name="s")

    @functools.partial(
        pl.kernel,
        mesh=mesh,
        out_type=jax.ShapeDtypeStruct((_N, _D), jnp.float32),
        scratch_types=[
            pltpu.VMEM((_BPW,), jnp.int32),
            pltpu.VMEM((_CH, _D), jnp.float32),
            pltpu.VMEM((_CH, _D), jnp.float32),
            pltpu.SemaphoreType.DMA,
            pltpu.SemaphoreType.DMA,
        ],
    )
    def gather(cb_hbm, idx_hbm, out_hbm, idx_v, buf0, buf1, sem0, sem1):
        wid = lax.axis_index("s") * _NC + lax.axis_index("c")
        base = wid * _BPW
        pltpu.sync_copy(idx_hbm.at[pl.ds(base, _BPW)], idx_v)
        bufs = (buf0, buf1)
        sems = (sem0, sem1)
        cps = []
        for c in range(_NCH):
            cps.append(pltpu.async_copy(
                cb_hbm.at[idx_v.at[pl.ds(c * _CH, _CH)]],
                bufs[c % 2], sems[c % 2]))
            if c >= 1:
                cps[c - 1].wait()
                pltpu.sync_copy(bufs[(c - 1) % 2],
                                out_hbm.at[pl.ds(base + (c - 1) * _CH, _CH)])
        cps[-1].wait()
        pltpu.sync_copy(bufs[(_NCH - 1) % 2],
                        out_hbm.at[pl.ds(base + (_NCH - 1) * _CH, _CH)])

    return gather


_gather_rows = _make_gather()


def kernel(x, codebooks):
    b, t, d = x.shape
    xf = x.reshape(_N, _D)
    cbt16 = jnp.transpose(codebooks, (0, 2, 1)).astype(jnp.bfloat16)

    r = xf
    idx_list = []
    ss_list = []
    quants = None
    for i in range(_DEPTH):
        rn = jnp.sum(r ** 2.0, axis=1, keepdims=True)
        cn = jnp.sum(codebooks[i] ** 2.0, axis=1)[None, :]
        idx = _argmin_call(r, rn, cbt16[i], cn,
                           "C" if i == 3 else "B")
        q = _gather_rows(codebooks[i], idx)
        idx_list.append(idx)
        if i < _DEPTH - 1:
            r, ss = _resid_call(r, q)
        else:
            quants, ss = _final_call(xf, r, q)
        ss_list.append(ss)

    denom = jnp.float32(_N * _D)
    losses = [jnp.sum(s[:, 0, 0]) / denom for s in ss_list]
    commitment_loss = jnp.mean(jnp.stack(losses))
    codes = jnp.stack(idx_list, axis=-1).reshape(b, t, _DEPTH)
    quants = quants.reshape(b, t, d)
    return quants, commitment_loss, codes
